# Initial kernel scaffold; baseline (speedup 1.0000x reference)
import functools

import jax
import jax.numpy as jnp
from jax import lax
from jax.experimental import pallas as pl
from jax.experimental.pallas import tpu as pltpu
from jax.experimental.pallas import tpu_sc as plsc

N = 10000
D = 128
E = 320000
NC = 2
NS = 16
EPW = E // (NC * NS)
K = 100
NCH = EPW // K
RPS = N // NS

_MESH = plsc.VectorSubcoreMesh(core_axis_name="c", subcore_axis_name="s")


@functools.partial(
    pl.kernel,
    mesh=_MESH,
    out_type=jax.ShapeDtypeStruct((NC, N, 16), jnp.float32),
    scratch_types=[
        pltpu.VMEM((NCH, K), jnp.int32),
        pltpu.VMEM((K, 16), jnp.float32),
        pltpu.VMEM_SHARED((N, 16), jnp.float32),
    ],
)
def _hist(dst_hbm, zeros16_hbm, ones16_hbm, out_hbm, dst_v, ones_v, deg_sp):
    c = lax.axis_index("c")
    s = lax.axis_index("s")
    rows = pl.ds(s * RPS, RPS)
    pltpu.sync_copy(zeros16_hbm.at[rows], deg_sp.at[rows])
    pltpu.sync_copy(dst_hbm.at[c, s], dst_v)
    pltpu.sync_copy(ones16_hbm, ones_v)
    plsc.subcore_barrier()

    @pl.loop(0, NCH)
    def _(j):
        pltpu.sync_copy(ones_v, deg_sp.at[dst_v.at[j]], add=True)

    plsc.subcore_barrier()
    pltpu.sync_copy(deg_sp.at[rows], out_hbm.at[c].at[rows])


@functools.partial(
    pl.kernel,
    mesh=_MESH,
    out_type=jax.ShapeDtypeStruct((NC, N, D), jnp.float32),
    scratch_types=[
        pltpu.VMEM((NCH, K), jnp.int32),
        pltpu.VMEM((NCH, K), jnp.int32),
        pltpu.VMEM((K, D), jnp.float32),
        pltpu.VMEM((K, D), jnp.float32),
        pltpu.VMEM_SHARED((N, D), jnp.float32),
        pltpu.SemaphoreType.DMA,
        pltpu.SemaphoreType.DMA,
    ],
)
def _agg(y_hbm, src_hbm, dst_hbm, zeros_hbm, out_hbm,
         src_v, dst_v, buf_a, buf_b, acc_sp, sem_a, sem_b):
    c = lax.axis_index("c")
    s = lax.axis_index("s")
    rows = pl.ds(s * RPS, RPS)
    pltpu.sync_copy(zeros_hbm.at[rows], acc_sp.at[rows])
    pltpu.sync_copy(src_hbm.at[c, s], src_v)
    pltpu.sync_copy(dst_hbm.at[c, s], dst_v)
    plsc.subcore_barrier()

    def start(j, buf, sem):
        pltpu.async_copy(y_hbm.at[src_v.at[j]], buf, sem)

    def wait(j, buf, sem):
        pltpu.make_async_copy(y_hbm.at[src_v.at[j]], buf, sem).wait()

    def scat(j, buf):
        pltpu.sync_copy(buf, acc_sp.at[dst_v.at[j]], add=True)

    start(0, buf_a, sem_a)
    start(1, buf_b, sem_b)

    @pl.loop(0, NCH - 2, step=2)
    def _(j):
        wait(j, buf_a, sem_a)
        scat(j, buf_a)
        start(j + 2, buf_a, sem_a)
        wait(j + 1, buf_b, sem_b)
        scat(j + 1, buf_b)
        start(j + 3, buf_b, sem_b)

    wait(NCH - 2, buf_a, sem_a)
    scat(NCH - 2, buf_a)
    wait(NCH - 1, buf_b, sem_b)
    scat(NCH - 1, buf_b)

    plsc.subcore_barrier()
    pltpu.sync_copy(acc_sp.at[rows], out_hbm.at[c].at[rows])


R = 2000
G = N // R

_F32 = jnp.float32
_HI = lax.Precision.HIGHEST


def _lin1_body(hist_ref, x_ref, w_ref, y_ref, dis_ref):
    deg = hist_ref[0, :, 0:1] + hist_ref[1, :, 0:1] + 1.0
    dis = lax.rsqrt(deg)
    dis_ref[...] = dis
    xw = jnp.dot(x_ref[...], w_ref[...],
                 preferred_element_type=_F32, precision=_HI)
    y_ref[...] = xw * dis


def _lin1(hist, x, w):
    return pl.pallas_call(
        _lin1_body,
        grid=(G,),
        in_specs=[
            pl.BlockSpec((NC, R, 16), lambda i: (0, i, 0)),
            pl.BlockSpec((R, D), lambda i: (i, 0)),
            pl.BlockSpec((D, D), lambda i: (0, 0)),
        ],
        out_specs=[
            pl.BlockSpec((R, D), lambda i: (i, 0)),
            pl.BlockSpec((R, 1), lambda i: (i, 0)),
        ],
        out_shape=[
            jax.ShapeDtypeStruct((N, D), _F32),
            jax.ShapeDtypeStruct((N, 1), _F32),
        ],
    )(hist, x, w)


def _comb1_body(agg_ref, y_ref, dis_ref, b_ref, w_ref, o_ref):
    t = (agg_ref[0] + agg_ref[1] + y_ref[...]) * dis_ref[...] + b_ref[...]
    h = jnp.maximum(t, 0.0)
    hw = jnp.dot(h, w_ref[...], preferred_element_type=_F32, precision=_HI)
    o_ref[...] = hw * dis_ref[...]


def _comb1(agg, y, dis, b, w):
    return pl.pallas_call(
        _comb1_body,
        grid=(G,),
        in_specs=[
            pl.BlockSpec((NC, R, D), lambda i: (0, i, 0)),
            pl.BlockSpec((R, D), lambda i: (i, 0)),
            pl.BlockSpec((R, 1), lambda i: (i, 0)),
            pl.BlockSpec((1, D), lambda i: (0, 0)),
            pl.BlockSpec((D, D), lambda i: (0, 0)),
        ],
        out_specs=pl.BlockSpec((R, D), lambda i: (i, 0)),
        out_shape=jax.ShapeDtypeStruct((N, D), _F32),
    )(agg, y, dis, b, w)


def _comb2_body(agg_ref, y_ref, dis_ref, b_ref, o_ref):
    o_ref[...] = (agg_ref[0] + agg_ref[1] + y_ref[...]) * dis_ref[...] \
        + b_ref[...]


def _comb2(agg, y, dis, b):
    return pl.pallas_call(
        _comb2_body,
        grid=(G,),
        in_specs=[
            pl.BlockSpec((NC, R, D), lambda i: (0, i, 0)),
            pl.BlockSpec((R, D), lambda i: (i, 0)),
            pl.BlockSpec((R, 1), lambda i: (i, 0)),
            pl.BlockSpec((1, D), lambda i: (0, 0)),
        ],
        out_specs=pl.BlockSpec((R, D), lambda i: (i, 0)),
        out_shape=jax.ShapeDtypeStruct((N, D), _F32),
    )(agg, y, dis, b)


def kernel(x, edge_index, W1, b1, W2, b2):
    src = edge_index[0].astype(jnp.int32).reshape(NC, NS, NCH, K)
    dst = edge_index[1].astype(jnp.int32).reshape(NC, NS, NCH, K)
    zeros_d = jnp.zeros((N, D), jnp.float32)
    zeros16 = jnp.zeros((N, 16), jnp.float32)
    ones16 = jnp.ones((K, 16), jnp.float32)

    hist = _hist(dst, zeros16, ones16)
    y1, dis = _lin1(hist, x, W1)
    agg1 = _agg(y1, src, dst, zeros_d)
    y2 = _comb1(agg1, y1, dis, b1.reshape(1, D), W2)
    agg2 = _agg(y2, src, dst, zeros_d)
    return _comb2(agg2, y2, dis, b2.reshape(1, D))


# speedup vs baseline: 24.4085x; 24.4085x over previous
import functools

import jax
import jax.numpy as jnp
from jax import lax
from jax.experimental import pallas as pl
from jax.experimental.pallas import tpu as pltpu
from jax.experimental.pallas import tpu_sc as plsc

N = 10000
D = 128
H = D // 2
E = 320000
NC = 2
NS = 16
K = 100
NCH = E // (NC * NS * K)
NCH2 = E // (NS * K)
NP = 10240
RPS = NP // NS


def _sc_mesh():
    return plsc.VectorSubcoreMesh(core_axis_name="c", subcore_axis_name="s")


def _hist_body(dst_hbm, zeros16_hbm, ones16_hbm, out_hbm, dst_v, ones_v,
               deg_sp):
    c = lax.axis_index("c")
    s = lax.axis_index("s")
    rows = pl.ds(s * RPS, RPS)
    pltpu.sync_copy(zeros16_hbm.at[rows], deg_sp.at[rows])
    pltpu.sync_copy(dst_hbm.at[c, s], dst_v)
    pltpu.sync_copy(ones16_hbm, ones_v)
    plsc.subcore_barrier()

    @pl.loop(0, NCH)
    def _(j):
        pltpu.sync_copy(ones_v, deg_sp.at[dst_v.at[j]], add=True)

    plsc.subcore_barrier()
    pltpu.sync_copy(deg_sp.at[rows], out_hbm.at[c].at[rows])


@functools.cache
def _hist():
    return pl.kernel(
        _hist_body,
        mesh=_sc_mesh(),
        compiler_params=pltpu.CompilerParams(use_tc_tiling_on_sc=False),
        out_type=jax.ShapeDtypeStruct((NC, NP, 16), jnp.float32),
        scratch_types=[
            pltpu.VMEM((NCH, K), jnp.int32),
            pltpu.VMEM((K, 16), jnp.float32),
            pltpu.VMEM_SHARED((NP, 16), jnp.float32),
        ],
    )


def _agg_body(y_hbm, src_hbm, dst_hbm, zeros_hbm, out_hbm,
              src_v, dst_v, buf_a, buf_b, acc_sp, sem_a, sem_b):
    c = lax.axis_index("c")
    s = lax.axis_index("s")
    rows = pl.ds(s * RPS, RPS)
    pltpu.sync_copy(zeros_hbm.at[rows], acc_sp.at[rows])
    pltpu.sync_copy(src_hbm.at[s], src_v)
    pltpu.sync_copy(dst_hbm.at[s], dst_v)
    plsc.subcore_barrier()

    my_y = y_hbm.at[c]

    def start(j, buf, sem):
        pltpu.async_copy(my_y.at[src_v.at[j]], buf, sem)

    def wait(j, buf, sem):
        pltpu.make_async_copy(my_y.at[src_v.at[j]], buf, sem).wait()

    def scat(j, buf):
        pltpu.sync_copy(buf, acc_sp.at[dst_v.at[j]], add=True)

    start(0, buf_a, sem_a)
    start(1, buf_b, sem_b)

    @pl.loop(0, NCH2 - 2, step=2)
    def _(j):
        wait(j, buf_a, sem_a)
        scat(j, buf_a)
        start(j + 2, buf_a, sem_a)
        wait(j + 1, buf_b, sem_b)
        scat(j + 1, buf_b)
        start(j + 3, buf_b, sem_b)

    wait(NCH2 - 2, buf_a, sem_a)
    scat(NCH2 - 2, buf_a)
    wait(NCH2 - 1, buf_b, sem_b)
    scat(NCH2 - 1, buf_b)

    plsc.subcore_barrier()
    pltpu.sync_copy(acc_sp.at[rows], out_hbm.at[c].at[rows])


@functools.cache
def _agg():
    return pl.kernel(
        _agg_body,
        mesh=_sc_mesh(),
        compiler_params=pltpu.CompilerParams(use_tc_tiling_on_sc=False),
        out_type=jax.ShapeDtypeStruct((NC, NP, H), jnp.float32),
        scratch_types=[
            pltpu.VMEM((NCH2, K), jnp.int32),
            pltpu.VMEM((NCH2, K), jnp.int32),
            pltpu.VMEM((K, H), jnp.float32),
            pltpu.VMEM((K, H), jnp.float32),
            pltpu.VMEM_SHARED((NP, H), jnp.float32),
            pltpu.SemaphoreType.DMA,
            pltpu.SemaphoreType.DMA,
        ],
    )


R = 2000
G = N // R

_F32 = jnp.float32
_HI = lax.Precision.HIGHEST


def _lin1_body(hist_ref, x_ref, w_ref, y_ref, dis_ref):
    deg = hist_ref[0, :, 0:1] + hist_ref[1, :, 0:1] + 1.0
    dis = lax.rsqrt(deg)
    dis_ref[...] = dis
    xw = jnp.dot(x_ref[...], w_ref[...],
                 preferred_element_type=_F32, precision=_HI)
    y = xw * dis
    y_ref[0] = y[:, 0:H]
    y_ref[1] = y[:, H:D]


def _lin1(hist, x, w):
    return pl.pallas_call(
        _lin1_body,
        grid=(G,),
        in_specs=[
            pl.BlockSpec((NC, R, 16), lambda i: (0, i, 0)),
            pl.BlockSpec((R, D), lambda i: (i, 0)),
            pl.BlockSpec((D, D), lambda i: (0, 0)),
        ],
        out_specs=[
            pl.BlockSpec((NC, R, H), lambda i: (0, i, 0)),
            pl.BlockSpec((R, 1), lambda i: (i, 0)),
        ],
        out_shape=[
            jax.ShapeDtypeStruct((NC, N, H), _F32),
            jax.ShapeDtypeStruct((N, 1), _F32),
        ],
    )(hist, x, w)


def _comb1_body(agg_ref, y_ref, dis_ref, b_ref, w_ref, o_ref):
    dis = dis_ref[...]
    h0 = jnp.maximum((agg_ref[0] + y_ref[0]) * dis + b_ref[:, 0:H], 0.0)
    h1 = jnp.maximum((agg_ref[1] + y_ref[1]) * dis + b_ref[:, H:D], 0.0)
    hw = (jnp.dot(h0, w_ref[0:H, :], preferred_element_type=_F32,
                  precision=_HI)
          + jnp.dot(h1, w_ref[H:D, :], preferred_element_type=_F32,
                    precision=_HI))
    y2 = hw * dis
    o_ref[0] = y2[:, 0:H]
    o_ref[1] = y2[:, H:D]


def _comb1(agg, y, dis, b, w):
    return pl.pallas_call(
        _comb1_body,
        grid=(G,),
        in_specs=[
            pl.BlockSpec((NC, R, H), lambda i: (0, i, 0)),
            pl.BlockSpec((NC, R, H), lambda i: (0, i, 0)),
            pl.BlockSpec((R, 1), lambda i: (i, 0)),
            pl.BlockSpec((1, D), lambda i: (0, 0)),
            pl.BlockSpec((D, D), lambda i: (0, 0)),
        ],
        out_specs=pl.BlockSpec((NC, R, H), lambda i: (0, i, 0)),
        out_shape=jax.ShapeDtypeStruct((NC, N, H), _F32),
    )(agg, y, dis, b, w)


def _comb2_body(agg_ref, y_ref, dis_ref, b_ref, o_ref):
    dis = dis_ref[...]
    o_ref[:, 0:H] = (agg_ref[0] + y_ref[0]) * dis + b_ref[:, 0:H]
    o_ref[:, H:D] = (agg_ref[1] + y_ref[1]) * dis + b_ref[:, H:D]


def _comb2(agg, y, dis, b):
    return pl.pallas_call(
        _comb2_body,
        grid=(G,),
        in_specs=[
            pl.BlockSpec((NC, R, H), lambda i: (0, i, 0)),
            pl.BlockSpec((NC, R, H), lambda i: (0, i, 0)),
            pl.BlockSpec((R, 1), lambda i: (i, 0)),
            pl.BlockSpec((1, D), lambda i: (0, 0)),
        ],
        out_specs=pl.BlockSpec((R, D), lambda i: (i, 0)),
        out_shape=jax.ShapeDtypeStruct((N, D), _F32),
    )(agg, y, dis, b)


def kernel(x, edge_index, W1, b1, W2, b2):
    srcf = edge_index[0].astype(jnp.int32)
    dstf = edge_index[1].astype(jnp.int32)
    src2 = srcf.reshape(NS, NCH2, K)
    dst2 = dstf.reshape(NS, NCH2, K)
    dsth = dstf.reshape(NC, NS, NCH, K)
    zeros_h = jnp.zeros((NP, H), jnp.float32)
    zeros16 = jnp.zeros((NP, 16), jnp.float32)
    ones16 = jnp.ones((K, 16), jnp.float32)

    hist = _hist()(dsth, zeros16, ones16)
    y1, dis = _lin1(hist, x, W1)
    agg1 = _agg()(y1, src2, dst2, zeros_h)
    y2 = _comb1(agg1, y1, dis, b1.reshape(1, D), W2)
    agg2 = _agg()(y2, src2, dst2, zeros_h)
    return _comb2(agg2, y2, dis, b2.reshape(1, D))


# speedup vs baseline: 28.0515x; 1.1493x over previous
import functools

import jax
import jax.numpy as jnp
from jax import lax
from jax.experimental import pallas as pl
from jax.experimental.pallas import tpu as pltpu
from jax.experimental.pallas import tpu_sc as plsc

N = 10000
D = 128
H = D // 2
E = 320000
NC = 2
NS = 16
K = 100
NCH = E // (NC * NS * K)
K2 = 125
NCH2 = E // (NS * K2)
NBUF = 4
NP = 10240
RPS = NP // NS


def _sc_mesh():
    return plsc.VectorSubcoreMesh(core_axis_name="c", subcore_axis_name="s")


def _hist_body(dst_hbm, zeros16_hbm, ones16_hbm, out_hbm, dst_v, ones_v,
               deg_sp):
    c = lax.axis_index("c")
    s = lax.axis_index("s")
    rows = pl.ds(s * RPS, RPS)
    pltpu.sync_copy(zeros16_hbm.at[rows], deg_sp.at[rows])
    pltpu.sync_copy(dst_hbm.at[c, s], dst_v)
    pltpu.sync_copy(ones16_hbm, ones_v)
    plsc.subcore_barrier()

    @pl.loop(0, NCH)
    def _(j):
        pltpu.sync_copy(ones_v, deg_sp.at[dst_v.at[j]], add=True)

    plsc.subcore_barrier()
    pltpu.sync_copy(deg_sp.at[rows], out_hbm.at[c].at[rows])


@functools.cache
def _hist():
    return pl.kernel(
        _hist_body,
        mesh=_sc_mesh(),
        compiler_params=pltpu.CompilerParams(use_tc_tiling_on_sc=False),
        out_type=jax.ShapeDtypeStruct((NC, NP, 16), jnp.float32),
        scratch_types=[
            pltpu.VMEM((NCH, K), jnp.int32),
            pltpu.VMEM((K, 16), jnp.float32),
            pltpu.VMEM_SHARED((NP, 16), jnp.float32),
        ],
    )


def _agg_body(y_hbm, src_hbm, dst_hbm, zeros_hbm, out_hbm,
              src_v, dst_v, buf0, buf1, buf2, buf3, acc_sp,
              gs0, gs1, gs2, gs3, ss0, ss1, ss2, ss3):
    c = lax.axis_index("c")
    s = lax.axis_index("s")
    rows = pl.ds(s * RPS, RPS)
    pltpu.sync_copy(zeros_hbm.at[rows], acc_sp.at[rows])
    pltpu.sync_copy(src_hbm.at[s], src_v)
    pltpu.sync_copy(dst_hbm.at[s], dst_v)
    plsc.subcore_barrier()

    my_y = y_hbm.at[c]
    bufs = (buf0, buf1, buf2, buf3)
    gsems = (gs0, gs1, gs2, gs3)
    ssems = (ss0, ss1, ss2, ss3)

    def gat_start(j, i):
        pltpu.async_copy(my_y.at[src_v.at[j]], bufs[i], gsems[i])

    def gat_wait(j, i):
        pltpu.make_async_copy(my_y.at[src_v.at[j]], bufs[i], gsems[i]).wait()

    def scat_start(j, i):
        pltpu.async_copy(bufs[i], acc_sp.at[dst_v.at[j]], ssems[i],
                         add=True)

    def scat_wait(j, i):
        pltpu.make_async_copy(bufs[i], acc_sp.at[dst_v.at[j]],
                              ssems[i]).wait()

    for i in range(NBUF):
        gat_start(i, i)

    @pl.loop(0, NCH2 - NBUF, step=NBUF)
    def _(j):
        for i in range(NBUF):
            gat_wait(j + i, i)
            scat_start(j + i, i)
        for i in range(NBUF):
            scat_wait(j + i, i)
            gat_start(j + NBUF + i, i)

    for i in range(NBUF):
        gat_wait(NCH2 - NBUF + i, i)
        scat_start(NCH2 - NBUF + i, i)
    for i in range(NBUF):
        scat_wait(NCH2 - NBUF + i, i)

    plsc.subcore_barrier()
    pltpu.sync_copy(acc_sp.at[rows], out_hbm.at[c].at[rows])


@functools.cache
def _agg():
    return pl.kernel(
        _agg_body,
        mesh=_sc_mesh(),
        compiler_params=pltpu.CompilerParams(use_tc_tiling_on_sc=False),
        out_type=jax.ShapeDtypeStruct((NC, NP, H), jnp.float32),
        scratch_types=[
            pltpu.VMEM((NCH2, K2), jnp.int32),
            pltpu.VMEM((NCH2, K2), jnp.int32),
            pltpu.VMEM((K2, H), jnp.float32),
            pltpu.VMEM((K2, H), jnp.float32),
            pltpu.VMEM((K2, H), jnp.float32),
            pltpu.VMEM((K2, H), jnp.float32),
            pltpu.VMEM_SHARED((NP, H), jnp.float32),
            pltpu.SemaphoreType.DMA,
            pltpu.SemaphoreType.DMA,
            pltpu.SemaphoreType.DMA,
            pltpu.SemaphoreType.DMA,
            pltpu.SemaphoreType.DMA,
            pltpu.SemaphoreType.DMA,
            pltpu.SemaphoreType.DMA,
            pltpu.SemaphoreType.DMA,
        ],
    )


R = 2000
G = N // R

_F32 = jnp.float32
_HI = lax.Precision.HIGHEST


def _lin1_body(hist_ref, x_ref, w_ref, y_ref, dis_ref):
    deg = hist_ref[0, :, 0:1] + hist_ref[1, :, 0:1] + 1.0
    dis = lax.rsqrt(deg)
    dis_ref[...] = dis
    xw = jnp.dot(x_ref[...], w_ref[...],
                 preferred_element_type=_F32, precision=_HI)
    y = xw * dis
    y_ref[0] = y[:, 0:H]
    y_ref[1] = y[:, H:D]


def _lin1(hist, x, w):
    return pl.pallas_call(
        _lin1_body,
        grid=(G,),
        in_specs=[
            pl.BlockSpec((NC, R, 16), lambda i: (0, i, 0)),
            pl.BlockSpec((R, D), lambda i: (i, 0)),
            pl.BlockSpec((D, D), lambda i: (0, 0)),
        ],
        out_specs=[
            pl.BlockSpec((NC, R, H), lambda i: (0, i, 0)),
            pl.BlockSpec((R, 1), lambda i: (i, 0)),
        ],
        out_shape=[
            jax.ShapeDtypeStruct((NC, N, H), _F32),
            jax.ShapeDtypeStruct((N, 1), _F32),
        ],
    )(hist, x, w)


def _comb1_body(agg_ref, y_ref, dis_ref, b_ref, w_ref, o_ref):
    dis = dis_ref[...]
    h0 = jnp.maximum((agg_ref[0] + y_ref[0]) * dis + b_ref[:, 0:H], 0.0)
    h1 = jnp.maximum((agg_ref[1] + y_ref[1]) * dis + b_ref[:, H:D], 0.0)
    hw = (jnp.dot(h0, w_ref[0:H, :], preferred_element_type=_F32,
                  precision=_HI)
          + jnp.dot(h1, w_ref[H:D, :], preferred_element_type=_F32,
                    precision=_HI))
    y2 = hw * dis
    o_ref[0] = y2[:, 0:H]
    o_ref[1] = y2[:, H:D]


def _comb1(agg, y, dis, b, w):
    return pl.pallas_call(
        _comb1_body,
        grid=(G,),
        in_specs=[
            pl.BlockSpec((NC, R, H), lambda i: (0, i, 0)),
            pl.BlockSpec((NC, R, H), lambda i: (0, i, 0)),
            pl.BlockSpec((R, 1), lambda i: (i, 0)),
            pl.BlockSpec((1, D), lambda i: (0, 0)),
            pl.BlockSpec((D, D), lambda i: (0, 0)),
        ],
        out_specs=pl.BlockSpec((NC, R, H), lambda i: (0, i, 0)),
        out_shape=jax.ShapeDtypeStruct((NC, N, H), _F32),
    )(agg, y, dis, b, w)


def _comb2_body(agg_ref, y_ref, dis_ref, b_ref, o_ref):
    dis = dis_ref[...]
    o_ref[:, 0:H] = (agg_ref[0] + y_ref[0]) * dis + b_ref[:, 0:H]
    o_ref[:, H:D] = (agg_ref[1] + y_ref[1]) * dis + b_ref[:, H:D]


def _comb2(agg, y, dis, b):
    return pl.pallas_call(
        _comb2_body,
        grid=(G,),
        in_specs=[
            pl.BlockSpec((NC, R, H), lambda i: (0, i, 0)),
            pl.BlockSpec((NC, R, H), lambda i: (0, i, 0)),
            pl.BlockSpec((R, 1), lambda i: (i, 0)),
            pl.BlockSpec((1, D), lambda i: (0, 0)),
        ],
        out_specs=pl.BlockSpec((R, D), lambda i: (i, 0)),
        out_shape=jax.ShapeDtypeStruct((N, D), _F32),
    )(agg, y, dis, b)


def kernel(x, edge_index, W1, b1, W2, b2):
    srcf = edge_index[0].astype(jnp.int32)
    dstf = edge_index[1].astype(jnp.int32)
    src2 = srcf.reshape(NS, NCH2, K2)
    dst2 = dstf.reshape(NS, NCH2, K2)
    dsth = dstf.reshape(NC, NS, NCH, K)
    zeros_h = jnp.zeros((NP, H), jnp.float32)
    zeros16 = jnp.zeros((NP, 16), jnp.float32)
    ones16 = jnp.ones((K, 16), jnp.float32)

    hist = _hist()(dsth, zeros16, ones16)
    y1, dis = _lin1(hist, x, W1)
    agg1 = _agg()(y1, src2, dst2, zeros_h)
    y2 = _comb1(agg1, y1, dis, b1.reshape(1, D), W2)
    agg2 = _agg()(y2, src2, dst2, zeros_h)
    return _comb2(agg2, y2, dis, b2.reshape(1, D))


# speedup vs baseline: 28.3801x; 1.0117x over previous
import functools

import jax
import jax.numpy as jnp
from jax import lax
from jax.experimental import pallas as pl
from jax.experimental.pallas import tpu as pltpu
from jax.experimental.pallas import tpu_sc as plsc

N = 10000
D = 128
H = D // 2
E = 320000
NC = 2
NS = 16
K = 125
NCH = E // (NS * K)
NCHH = NCH // NC
NBUF = 4
HFD = 8
NP = 10240
RPS = NP // NS


def _sc_mesh():
    return plsc.VectorSubcoreMesh(core_axis_name="c", subcore_axis_name="s")


def _hist_body(dst_hbm, zeros16_hbm, ones16_hbm, out_hbm, dst_v, ones_v,
               deg_sp, sem):
    c = lax.axis_index("c")
    s = lax.axis_index("s")
    rows = pl.ds(s * RPS, RPS)
    pltpu.sync_copy(zeros16_hbm.at[rows], deg_sp.at[rows])
    pltpu.sync_copy(dst_hbm.at[s, pl.ds(c * NCHH, NCHH)], dst_v)
    pltpu.sync_copy(ones16_hbm, ones_v)
    plsc.subcore_barrier()

    @pl.loop(0, NCHH, step=HFD)
    def _(j):
        for i in range(HFD):
            pltpu.async_copy(ones_v, deg_sp.at[dst_v.at[j + i]], sem,
                             add=True)
        for i in range(HFD):
            pltpu.make_async_copy(ones_v, deg_sp.at[dst_v.at[j + i]],
                                  sem).wait()

    plsc.subcore_barrier()
    pltpu.sync_copy(deg_sp.at[rows], out_hbm.at[c].at[rows])


@functools.cache
def _hist():
    return pl.kernel(
        _hist_body,
        mesh=_sc_mesh(),
        compiler_params=pltpu.CompilerParams(use_tc_tiling_on_sc=False),
        out_type=jax.ShapeDtypeStruct((NC, NP, 16), jnp.float32),
        scratch_types=[
            pltpu.VMEM((NCHH, K), jnp.int32),
            pltpu.VMEM((K, 16), jnp.float32),
            pltpu.VMEM_SHARED((NP, 16), jnp.float32),
            pltpu.SemaphoreType.DMA,
        ],
    )


def _agg_body(y_hbm, src_hbm, dst_hbm, zeros_hbm, out_hbm,
              src_v, dst_v, *rest):
    bufs = rest[:NBUF]
    acc_sp = rest[NBUF]
    gsems = rest[NBUF + 1:2 * NBUF + 1]
    ssems = rest[2 * NBUF + 1:3 * NBUF + 1]
    c = lax.axis_index("c")
    s = lax.axis_index("s")
    rows = pl.ds(s * RPS, RPS)
    pltpu.sync_copy(zeros_hbm.at[rows], acc_sp.at[rows])
    pltpu.sync_copy(src_hbm.at[s], src_v)
    pltpu.sync_copy(dst_hbm.at[s], dst_v)
    plsc.subcore_barrier()

    my_y = y_hbm.at[c]

    def gat_start(j, i):
        pltpu.async_copy(my_y.at[src_v.at[j]], bufs[i], gsems[i])

    def gat_wait(j, i):
        pltpu.make_async_copy(my_y.at[src_v.at[j]], bufs[i], gsems[i]).wait()

    def scat_start(j, i):
        pltpu.async_copy(bufs[i], acc_sp.at[dst_v.at[j]], ssems[i],
                         add=True)

    def scat_wait(j, i):
        pltpu.make_async_copy(bufs[i], acc_sp.at[dst_v.at[j]],
                              ssems[i]).wait()

    for i in range(NBUF):
        gat_start(i, i)

    @pl.loop(0, NCH - NBUF, step=NBUF)
    def _(j):
        for i in range(NBUF):
            gat_wait(j + i, i)
            scat_start(j + i, i)
        for i in range(NBUF):
            scat_wait(j + i, i)
            gat_start(j + NBUF + i, i)

    for i in range(NBUF):
        gat_wait(NCH - NBUF + i, i)
        scat_start(NCH - NBUF + i, i)
    for i in range(NBUF):
        scat_wait(NCH - NBUF + i, i)

    plsc.subcore_barrier()
    pltpu.sync_copy(acc_sp.at[rows], out_hbm.at[c].at[rows])


@functools.cache
def _agg():
    return pl.kernel(
        _agg_body,
        mesh=_sc_mesh(),
        compiler_params=pltpu.CompilerParams(use_tc_tiling_on_sc=False),
        out_type=jax.ShapeDtypeStruct((NC, NP, H), jnp.float32),
        scratch_types=(
            [pltpu.VMEM((NCH, K), jnp.int32),
             pltpu.VMEM((NCH, K), jnp.int32)]
            + [pltpu.VMEM((K, H), jnp.float32) for _ in range(NBUF)]
            + [pltpu.VMEM_SHARED((NP, H), jnp.float32)]
            + [pltpu.SemaphoreType.DMA for _ in range(2 * NBUF)]
        ),
    )


R = 2000
G = N // R

_F32 = jnp.float32
_HI = lax.Precision.HIGHEST


def _lin1_body(hist_ref, x_ref, w_ref, y_ref, dis_ref):
    deg = hist_ref[0, :, 0:1] + hist_ref[1, :, 0:1] + 1.0
    dis = lax.rsqrt(deg)
    dis_ref[...] = dis
    xw = jnp.dot(x_ref[...], w_ref[...],
                 preferred_element_type=_F32, precision=_HI)
    y = xw * dis
    y_ref[0] = y[:, 0:H]
    y_ref[1] = y[:, H:D]


def _lin1(hist, x, w):
    return pl.pallas_call(
        _lin1_body,
        grid=(G,),
        in_specs=[
            pl.BlockSpec((NC, R, 16), lambda i: (0, i, 0)),
            pl.BlockSpec((R, D), lambda i: (i, 0)),
            pl.BlockSpec((D, D), lambda i: (0, 0)),
        ],
        out_specs=[
            pl.BlockSpec((NC, R, H), lambda i: (0, i, 0)),
            pl.BlockSpec((R, 1), lambda i: (i, 0)),
        ],
        out_shape=[
            jax.ShapeDtypeStruct((NC, N, H), _F32),
            jax.ShapeDtypeStruct((N, 1), _F32),
        ],
    )(hist, x, w)


def _comb1_body(agg_ref, y_ref, dis_ref, b_ref, w_ref, o_ref):
    dis = dis_ref[...]
    h0 = jnp.maximum((agg_ref[0] + y_ref[0]) * dis + b_ref[:, 0:H], 0.0)
    h1 = jnp.maximum((agg_ref[1] + y_ref[1]) * dis + b_ref[:, H:D], 0.0)
    hw = (jnp.dot(h0, w_ref[0:H, :], preferred_element_type=_F32,
                  precision=_HI)
          + jnp.dot(h1, w_ref[H:D, :], preferred_element_type=_F32,
                    precision=_HI))
    y2 = hw * dis
    o_ref[0] = y2[:, 0:H]
    o_ref[1] = y2[:, H:D]


def _comb1(agg, y, dis, b, w):
    return pl.pallas_call(
        _comb1_body,
        grid=(G,),
        in_specs=[
            pl.BlockSpec((NC, R, H), lambda i: (0, i, 0)),
            pl.BlockSpec((NC, R, H), lambda i: (0, i, 0)),
            pl.BlockSpec((R, 1), lambda i: (i, 0)),
            pl.BlockSpec((1, D), lambda i: (0, 0)),
            pl.BlockSpec((D, D), lambda i: (0, 0)),
        ],
        out_specs=pl.BlockSpec((NC, R, H), lambda i: (0, i, 0)),
        out_shape=jax.ShapeDtypeStruct((NC, N, H), _F32),
    )(agg, y, dis, b, w)


def _comb2_body(agg_ref, y_ref, dis_ref, b_ref, o_ref):
    dis = dis_ref[...]
    o_ref[:, 0:H] = (agg_ref[0] + y_ref[0]) * dis + b_ref[:, 0:H]
    o_ref[:, H:D] = (agg_ref[1] + y_ref[1]) * dis + b_ref[:, H:D]


def _comb2(agg, y, dis, b):
    return pl.pallas_call(
        _comb2_body,
        grid=(G,),
        in_specs=[
            pl.BlockSpec((NC, R, H), lambda i: (0, i, 0)),
            pl.BlockSpec((NC, R, H), lambda i: (0, i, 0)),
            pl.BlockSpec((R, 1), lambda i: (i, 0)),
            pl.BlockSpec((1, D), lambda i: (0, 0)),
        ],
        out_specs=pl.BlockSpec((R, D), lambda i: (i, 0)),
        out_shape=jax.ShapeDtypeStruct((N, D), _F32),
    )(agg, y, dis, b)


def kernel(x, edge_index, W1, b1, W2, b2):
    src = edge_index[0].astype(jnp.int32).reshape(NS, NCH, K)
    dst = edge_index[1].astype(jnp.int32).reshape(NS, NCH, K)
    zeros_h = jnp.zeros((NP, H), jnp.float32)
    zeros16 = jnp.zeros((NP, 16), jnp.float32)
    ones16 = jnp.ones((K, 16), jnp.float32)

    hist = _hist()(dst, zeros16, ones16)
    y1, dis = _lin1(hist, x, W1)
    agg1 = _agg()(y1, src, dst, zeros_h)
    y2 = _comb1(agg1, y1, dis, b1.reshape(1, D), W2)
    agg2 = _agg()(y2, src, dst, zeros_h)
    return _comb2(agg2, y2, dis, b2.reshape(1, D))


# speedup vs baseline: 28.8432x; 1.0163x over previous
import functools

import jax
import jax.numpy as jnp
from jax import lax
from jax.experimental import pallas as pl
from jax.experimental.pallas import tpu as pltpu
from jax.experimental.pallas import tpu_sc as plsc

N = 10000
D = 128
H = D // 2
E = 320000
NC = 2
NS = 16
K = 125
NCH = E // (NS * K)
NCHH = NCH // NC
NBUF = 5
HFD = 8
NP = 10240
RPS = NP // NS


def _sc_mesh():
    return plsc.VectorSubcoreMesh(core_axis_name="c", subcore_axis_name="s")


def _hist_body(dst_hbm, zeros16_hbm, ones16_hbm, out_hbm, dst_v, ones_v,
               deg_sp, sem):
    c = lax.axis_index("c")
    s = lax.axis_index("s")
    rows = pl.ds(s * RPS, RPS)
    pltpu.sync_copy(zeros16_hbm.at[rows], deg_sp.at[rows])
    pltpu.sync_copy(dst_hbm.at[s, pl.ds(c * NCHH, NCHH)], dst_v)
    pltpu.sync_copy(ones16_hbm, ones_v)
    plsc.subcore_barrier()

    @pl.loop(0, NCHH, step=HFD)
    def _(j):
        for i in range(HFD):
            pltpu.async_copy(ones_v, deg_sp.at[dst_v.at[j + i]], sem,
                             add=True)
        for i in range(HFD):
            pltpu.make_async_copy(ones_v, deg_sp.at[dst_v.at[j + i]],
                                  sem).wait()

    plsc.subcore_barrier()
    pltpu.sync_copy(deg_sp.at[rows], out_hbm.at[c].at[rows])


@functools.cache
def _hist():
    return pl.kernel(
        _hist_body,
        mesh=_sc_mesh(),
        compiler_params=pltpu.CompilerParams(use_tc_tiling_on_sc=False),
        out_type=jax.ShapeDtypeStruct((NC, NP, 16), jnp.float32),
        scratch_types=[
            pltpu.VMEM((NCHH, K), jnp.int32),
            pltpu.VMEM((K, 16), jnp.float32),
            pltpu.VMEM_SHARED((NP, 16), jnp.float32),
            pltpu.SemaphoreType.DMA,
        ],
    )


def _agg_body(y_hbm, src_hbm, dst_hbm, zeros_hbm, out_hbm,
              src_v, dst_v, *rest):
    bufs = rest[:NBUF]
    acc_sp = rest[NBUF]
    gsems = rest[NBUF + 1:2 * NBUF + 1]
    ssems = rest[2 * NBUF + 1:3 * NBUF + 1]
    c = lax.axis_index("c")
    s = lax.axis_index("s")
    rows = pl.ds(s * RPS, RPS)
    pltpu.sync_copy(zeros_hbm.at[rows], acc_sp.at[rows])
    pltpu.sync_copy(src_hbm.at[s], src_v)
    pltpu.sync_copy(dst_hbm.at[s], dst_v)
    plsc.subcore_barrier()

    my_y = y_hbm.at[c]

    def gat_start(j, i):
        pltpu.async_copy(my_y.at[src_v.at[j]], bufs[i], gsems[i])

    def gat_wait(j, i):
        pltpu.make_async_copy(my_y.at[src_v.at[j]], bufs[i], gsems[i]).wait()

    def scat_start(j, i):
        pltpu.async_copy(bufs[i], acc_sp.at[dst_v.at[j]], ssems[i],
                         add=True)

    def scat_wait(j, i):
        pltpu.make_async_copy(bufs[i], acc_sp.at[dst_v.at[j]],
                              ssems[i]).wait()

    for i in range(NBUF):
        gat_start(i, i)

    @pl.loop(0, NCH - NBUF, step=NBUF)
    def _(j):
        for i in range(NBUF):
            gat_wait(j + i, i)
            scat_start(j + i, i)
        for i in range(NBUF):
            scat_wait(j + i, i)
            gat_start(j + NBUF + i, i)

    for i in range(NBUF):
        gat_wait(NCH - NBUF + i, i)
        scat_start(NCH - NBUF + i, i)
    for i in range(NBUF):
        scat_wait(NCH - NBUF + i, i)

    plsc.subcore_barrier()
    pltpu.sync_copy(acc_sp.at[rows], out_hbm.at[c].at[rows])


@functools.cache
def _agg():
    return pl.kernel(
        _agg_body,
        mesh=_sc_mesh(),
        compiler_params=pltpu.CompilerParams(use_tc_tiling_on_sc=False),
        out_type=jax.ShapeDtypeStruct((NC, NP, H), jnp.float32),
        scratch_types=(
            [pltpu.VMEM((NCH, K), jnp.int32),
             pltpu.VMEM((NCH, K), jnp.int32)]
            + [pltpu.VMEM((K, H), jnp.float32) for _ in range(NBUF)]
            + [pltpu.VMEM_SHARED((NP, H), jnp.float32)]
            + [pltpu.SemaphoreType.DMA for _ in range(2 * NBUF)]
        ),
    )


R = 2000
G = N // R

_F32 = jnp.float32
_HI = lax.Precision.HIGHEST


def _mm1_body(x_ref, w_ref, xw_ref):
    xw_ref[...] = jnp.dot(x_ref[...], w_ref[...],
                          preferred_element_type=_F32, precision=_HI)


def _mm1(x, w):
    return pl.pallas_call(
        _mm1_body,
        grid=(G,),
        in_specs=[
            pl.BlockSpec((R, D), lambda i: (i, 0)),
            pl.BlockSpec((D, D), lambda i: (0, 0)),
        ],
        out_specs=pl.BlockSpec((R, D), lambda i: (i, 0)),
        out_shape=jax.ShapeDtypeStruct((N, D), _F32),
    )(x, w)


def _scale1_body(hist_ref, xw_ref, y_ref, dis_ref):
    deg = hist_ref[0, :, 0:1] + hist_ref[1, :, 0:1] + 1.0
    dis = lax.rsqrt(deg)
    dis_ref[...] = dis
    y = xw_ref[...] * dis
    y_ref[0] = y[:, 0:H]
    y_ref[1] = y[:, H:D]


def _scale1(hist, xw):
    return pl.pallas_call(
        _scale1_body,
        grid=(G,),
        in_specs=[
            pl.BlockSpec((NC, R, 16), lambda i: (0, i, 0)),
            pl.BlockSpec((R, D), lambda i: (i, 0)),
        ],
        out_specs=[
            pl.BlockSpec((NC, R, H), lambda i: (0, i, 0)),
            pl.BlockSpec((R, 1), lambda i: (i, 0)),
        ],
        out_shape=[
            jax.ShapeDtypeStruct((NC, N, H), _F32),
            jax.ShapeDtypeStruct((N, 1), _F32),
        ],
    )(hist, xw)


def _comb1_body(agg_ref, y_ref, dis_ref, b_ref, w_ref, o_ref):
    dis = dis_ref[...]
    h0 = jnp.maximum((agg_ref[0] + y_ref[0]) * dis + b_ref[:, 0:H], 0.0)
    h1 = jnp.maximum((agg_ref[1] + y_ref[1]) * dis + b_ref[:, H:D], 0.0)
    hw = (jnp.dot(h0, w_ref[0:H, :], preferred_element_type=_F32,
                  precision=_HI)
          + jnp.dot(h1, w_ref[H:D, :], preferred_element_type=_F32,
                    precision=_HI))
    y2 = hw * dis
    o_ref[0] = y2[:, 0:H]
    o_ref[1] = y2[:, H:D]


def _comb1(agg, y, dis, b, w):
    return pl.pallas_call(
        _comb1_body,
        grid=(G,),
        in_specs=[
            pl.BlockSpec((NC, R, H), lambda i: (0, i, 0)),
            pl.BlockSpec((NC, R, H), lambda i: (0, i, 0)),
            pl.BlockSpec((R, 1), lambda i: (i, 0)),
            pl.BlockSpec((1, D), lambda i: (0, 0)),
            pl.BlockSpec((D, D), lambda i: (0, 0)),
        ],
        out_specs=pl.BlockSpec((NC, R, H), lambda i: (0, i, 0)),
        out_shape=jax.ShapeDtypeStruct((NC, N, H), _F32),
    )(agg, y, dis, b, w)


def _comb2_body(agg_ref, y_ref, dis_ref, b_ref, o_ref):
    dis = dis_ref[...]
    o_ref[:, 0:H] = (agg_ref[0] + y_ref[0]) * dis + b_ref[:, 0:H]
    o_ref[:, H:D] = (agg_ref[1] + y_ref[1]) * dis + b_ref[:, H:D]


def _comb2(agg, y, dis, b):
    return pl.pallas_call(
        _comb2_body,
        grid=(G,),
        in_specs=[
            pl.BlockSpec((NC, R, H), lambda i: (0, i, 0)),
            pl.BlockSpec((NC, R, H), lambda i: (0, i, 0)),
            pl.BlockSpec((R, 1), lambda i: (i, 0)),
            pl.BlockSpec((1, D), lambda i: (0, 0)),
        ],
        out_specs=pl.BlockSpec((R, D), lambda i: (i, 0)),
        out_shape=jax.ShapeDtypeStruct((N, D), _F32),
    )(agg, y, dis, b)


def kernel(x, edge_index, W1, b1, W2, b2):
    src = edge_index[0].astype(jnp.int32).reshape(NS, NCH, K)
    dst = edge_index[1].astype(jnp.int32).reshape(NS, NCH, K)
    zeros_h = jnp.zeros((NP, H), jnp.float32)
    zeros16 = jnp.zeros((NP, 16), jnp.float32)
    ones16 = jnp.ones((K, 16), jnp.float32)

    hist = _hist()(dst, zeros16, ones16)
    xw1 = _mm1(x, W1)
    y1, dis = _scale1(hist, xw1)
    agg1 = _agg()(y1, src, dst, zeros_h)
    y2 = _comb1(agg1, y1, dis, b1.reshape(1, D), W2)
    agg2 = _agg()(y2, src, dst, zeros_h)
    return _comb2(agg2, y2, dis, b2.reshape(1, D))


# speedup vs baseline: 29.3815x; 1.0187x over previous
import functools

import jax
import jax.numpy as jnp
from jax import lax
from jax.experimental import pallas as pl
from jax.experimental.pallas import tpu as pltpu
from jax.experimental.pallas import tpu_sc as plsc

N = 10000
D = 128
H = D // 2
E = 320000
NC = 2
NS = 16
K = 125
NCH = E // (NS * K)
NCHH = NCH // NC
NBUF = 5
HFD = 8
NP = 10240
RPS = NP // NS


def _sc_mesh():
    return plsc.VectorSubcoreMesh(core_axis_name="c", subcore_axis_name="s")


def _hist_body(dst_hbm, out_hbm, dst_v, ones_v, deg_sp, sem):
    c = lax.axis_index("c")
    s = lax.axis_index("s")
    rows = pl.ds(s * RPS, RPS)
    pltpu.sync_copy(dst_hbm.at[s, pl.ds(c * NCHH, NCHH)], dst_v)

    @pl.loop(0, K)
    def _(i):
        ones_v[i] = jnp.zeros((16,), jnp.float32)

    @pl.loop(0, RPS - K, step=K)
    def _(r):
        pltpu.sync_copy(ones_v, deg_sp.at[pl.ds(s * RPS + r, K)])
    pltpu.sync_copy(ones_v.at[pl.ds(0, RPS - (RPS // K) * K)],
                    deg_sp.at[pl.ds(s * RPS + (RPS // K) * K,
                                    RPS - (RPS // K) * K)])

    @pl.loop(0, K)
    def _(i):
        ones_v[i] = jnp.full((16,), 1.0, jnp.float32)

    plsc.subcore_barrier()

    @pl.loop(0, NCHH, step=HFD)
    def _(j):
        for i in range(HFD):
            pltpu.async_copy(ones_v, deg_sp.at[dst_v.at[j + i]], sem,
                             add=True)
        for i in range(HFD):
            pltpu.make_async_copy(ones_v, deg_sp.at[dst_v.at[j + i]],
                                  sem).wait()

    plsc.subcore_barrier()
    pltpu.sync_copy(deg_sp.at[rows], out_hbm.at[c].at[rows])


@functools.cache
def _hist():
    return pl.kernel(
        _hist_body,
        mesh=_sc_mesh(),
        compiler_params=pltpu.CompilerParams(use_tc_tiling_on_sc=False),
        out_type=jax.ShapeDtypeStruct((NC, NP, 16), jnp.float32),
        scratch_types=[
            pltpu.VMEM((NCHH, K), jnp.int32),
            pltpu.VMEM((K, 16), jnp.float32),
            pltpu.VMEM_SHARED((NP, 16), jnp.float32),
            pltpu.SemaphoreType.DMA,
        ],
    )


def _agg_body(y_hbm, src_hbm, dst_hbm, out_hbm, src_v, dst_v, *rest):
    bufs = rest[:NBUF]
    acc_sp = rest[NBUF]
    gsems = rest[NBUF + 1:2 * NBUF + 1]
    ssems = rest[2 * NBUF + 1:3 * NBUF + 1]
    c = lax.axis_index("c")
    s = lax.axis_index("s")
    rows = pl.ds(s * RPS, RPS)
    pltpu.sync_copy(src_hbm.at[s], src_v)
    pltpu.sync_copy(dst_hbm.at[s], dst_v)

    zb = rest[0]

    @pl.loop(0, K)
    def _(i):
        for q in range(H // 16):
            zb[i, pl.ds(16 * q, 16)] = jnp.zeros((16,), jnp.float32)

    @pl.loop(0, RPS - K, step=K)
    def _(r):
        pltpu.sync_copy(zb, acc_sp.at[pl.ds(s * RPS + r, K)])
    pltpu.sync_copy(zb.at[pl.ds(0, RPS - (RPS // K) * K)],
                    acc_sp.at[pl.ds(s * RPS + (RPS // K) * K,
                                    RPS - (RPS // K) * K)])
    plsc.subcore_barrier()

    my_y = y_hbm.at[c]

    def gat_start(j, i):
        pltpu.async_copy(my_y.at[src_v.at[j]], bufs[i], gsems[i])

    def gat_wait(j, i):
        pltpu.make_async_copy(my_y.at[src_v.at[j]], bufs[i], gsems[i]).wait()

    def scat_start(j, i):
        pltpu.async_copy(bufs[i], acc_sp.at[dst_v.at[j]], ssems[i],
                         add=True)

    def scat_wait(j, i):
        pltpu.make_async_copy(bufs[i], acc_sp.at[dst_v.at[j]],
                              ssems[i]).wait()

    for i in range(NBUF):
        gat_start(i, i)

    @pl.loop(0, NCH - NBUF, step=NBUF)
    def _(j):
        for i in range(NBUF):
            gat_wait(j + i, i)
            scat_start(j + i, i)
        for i in range(NBUF):
            scat_wait(j + i, i)
            gat_start(j + NBUF + i, i)

    for i in range(NBUF):
        gat_wait(NCH - NBUF + i, i)
        scat_start(NCH - NBUF + i, i)
    for i in range(NBUF):
        scat_wait(NCH - NBUF + i, i)

    plsc.subcore_barrier()
    pltpu.sync_copy(acc_sp.at[rows], out_hbm.at[c].at[rows])


@functools.cache
def _agg():
    return pl.kernel(
        _agg_body,
        mesh=_sc_mesh(),
        compiler_params=pltpu.CompilerParams(use_tc_tiling_on_sc=False),
        out_type=jax.ShapeDtypeStruct((NC, NP, H), jnp.float32),
        scratch_types=(
            [pltpu.VMEM((NCH, K), jnp.int32),
             pltpu.VMEM((NCH, K), jnp.int32)]
            + [pltpu.VMEM((K, H), jnp.float32) for _ in range(NBUF)]
            + [pltpu.VMEM_SHARED((NP, H), jnp.float32)]
            + [pltpu.SemaphoreType.DMA for _ in range(2 * NBUF)]
        ),
    )


R = 2000
G = N // R

_F32 = jnp.float32
_HI = lax.Precision.HIGHEST


def _mm1_body(x_ref, w_ref, xw_ref):
    xw_ref[...] = jnp.dot(x_ref[...], w_ref[...],
                          preferred_element_type=_F32, precision=_HI)


def _mm1(x, w):
    return pl.pallas_call(
        _mm1_body,
        grid=(G,),
        in_specs=[
            pl.BlockSpec((R, D), lambda i: (i, 0)),
            pl.BlockSpec((D, D), lambda i: (0, 0)),
        ],
        out_specs=pl.BlockSpec((R, D), lambda i: (i, 0)),
        out_shape=jax.ShapeDtypeStruct((N, D), _F32),
    )(x, w)


def _scale1_body(hist_ref, xw_ref, y_ref, dis_ref):
    deg = hist_ref[0, :, 0:1] + hist_ref[1, :, 0:1] + 1.0
    dis = lax.rsqrt(deg)
    dis_ref[...] = dis
    y = xw_ref[...] * dis
    y_ref[0] = y[:, 0:H]
    y_ref[1] = y[:, H:D]


def _scale1(hist, xw):
    return pl.pallas_call(
        _scale1_body,
        grid=(G,),
        in_specs=[
            pl.BlockSpec((NC, R, 16), lambda i: (0, i, 0)),
            pl.BlockSpec((R, D), lambda i: (i, 0)),
        ],
        out_specs=[
            pl.BlockSpec((NC, R, H), lambda i: (0, i, 0)),
            pl.BlockSpec((R, 1), lambda i: (i, 0)),
        ],
        out_shape=[
            jax.ShapeDtypeStruct((NC, N, H), _F32),
            jax.ShapeDtypeStruct((N, 1), _F32),
        ],
    )(hist, xw)


def _comb1_body(agg_ref, y_ref, dis_ref, b_ref, w_ref, o_ref):
    dis = dis_ref[...]
    h0 = jnp.maximum((agg_ref[0] + y_ref[0]) * dis + b_ref[:, 0:H], 0.0)
    h1 = jnp.maximum((agg_ref[1] + y_ref[1]) * dis + b_ref[:, H:D], 0.0)
    hw = (jnp.dot(h0, w_ref[0:H, :], preferred_element_type=_F32,
                  precision=_HI)
          + jnp.dot(h1, w_ref[H:D, :], preferred_element_type=_F32,
                    precision=_HI))
    y2 = hw * dis
    o_ref[0] = y2[:, 0:H]
    o_ref[1] = y2[:, H:D]


def _comb1(agg, y, dis, b, w):
    return pl.pallas_call(
        _comb1_body,
        grid=(G,),
        in_specs=[
            pl.BlockSpec((NC, R, H), lambda i: (0, i, 0)),
            pl.BlockSpec((NC, R, H), lambda i: (0, i, 0)),
            pl.BlockSpec((R, 1), lambda i: (i, 0)),
            pl.BlockSpec((1, D), lambda i: (0, 0)),
            pl.BlockSpec((D, D), lambda i: (0, 0)),
        ],
        out_specs=pl.BlockSpec((NC, R, H), lambda i: (0, i, 0)),
        out_shape=jax.ShapeDtypeStruct((NC, N, H), _F32),
    )(agg, y, dis, b, w)


def _comb2_body(agg_ref, y_ref, dis_ref, b_ref, o_ref):
    dis = dis_ref[...]
    o_ref[:, 0:H] = (agg_ref[0] + y_ref[0]) * dis + b_ref[:, 0:H]
    o_ref[:, H:D] = (agg_ref[1] + y_ref[1]) * dis + b_ref[:, H:D]


def _comb2(agg, y, dis, b):
    return pl.pallas_call(
        _comb2_body,
        grid=(G,),
        in_specs=[
            pl.BlockSpec((NC, R, H), lambda i: (0, i, 0)),
            pl.BlockSpec((NC, R, H), lambda i: (0, i, 0)),
            pl.BlockSpec((R, 1), lambda i: (i, 0)),
            pl.BlockSpec((1, D), lambda i: (0, 0)),
        ],
        out_specs=pl.BlockSpec((R, D), lambda i: (i, 0)),
        out_shape=jax.ShapeDtypeStruct((N, D), _F32),
    )(agg, y, dis, b)


def kernel(x, edge_index, W1, b1, W2, b2):
    src = edge_index[0].astype(jnp.int32).reshape(NS, NCH, K)
    dst = edge_index[1].astype(jnp.int32).reshape(NS, NCH, K)
    hist = _hist()(dst)
    xw1 = _mm1(x, W1)
    y1, dis = _scale1(hist, xw1)
    agg1 = _agg()(y1, src, dst)
    y2 = _comb1(agg1, y1, dis, b1.reshape(1, D), W2)
    agg2 = _agg()(y2, src, dst)
    return _comb2(agg2, y2, dis, b2.reshape(1, D))
